# Initial kernel scaffold; baseline (speedup 1.0000x reference)
import jax
import jax.numpy as jnp
from jax.experimental import pallas as pl


def kernel(points, W1, b1, W2, b2, W3, b3):
    raise NotImplementedError("write your pallas kernel here")


# speedup vs baseline: 1463.5886x; 1463.5886x over previous
import functools

import jax
import jax.numpy as jnp
from jax.experimental import pallas as pl
from jax.experimental.pallas import tpu as pltpu

_N = 1024
_HIGH = jax.lax.Precision.HIGHEST


def _gcn_body(p_ref, pt_ref, w1_ref, b1_ref, w2_ref, b2_ref, w3_ref, b3_ref,
              out_ref):
    p = p_ref[0]
    pt = pt_ref[0]
    px_c = p[:, 0:1]
    py_c = p[:, 1:2]
    px_r = pt[0:1, :]
    py_r = pt[1:2, :]

    dx = px_c - px_r
    dy = py_c - py_r
    dist = jnp.sqrt(dx * dx + dy * dy)
    a = (dist < 1.0).astype(jnp.float32)

    deg_c = jnp.sum(a, axis=1, keepdims=True)
    deg_r = jnp.sum(a, axis=0, keepdims=True)
    r_c = 1.0 / jnp.sqrt(deg_c)
    r_r = 1.0 / jnp.sqrt(deg_r)
    nrm = a * r_c * r_r

    xw1 = px_c * w1_ref[0:1, :] + py_c * w1_ref[1:2, :]
    h1 = jax.nn.relu(
        jnp.dot(nrm, xw1, preferred_element_type=jnp.float32,
                precision=_HIGH) + b1_ref[0:1, :])
    xw2 = jnp.dot(h1, w2_ref[...], preferred_element_type=jnp.float32,
                  precision=_HIGH)
    h2 = jax.nn.relu(
        jnp.dot(nrm, xw2, preferred_element_type=jnp.float32,
                precision=_HIGH) + b2_ref[0:1, :])
    xw3 = jnp.dot(h2, w3_ref[...], preferred_element_type=jnp.float32,
                  precision=_HIGH)
    out_ref[0] = jnp.dot(nrm, xw3, preferred_element_type=jnp.float32,
                         precision=_HIGH) + b3_ref[0:1, :]


@jax.jit
def kernel(points, W1, b1, W2, b2, W3, b3):
    bs, n, _ = points.shape
    pt = jnp.transpose(points, (0, 2, 1))
    d3 = W3.shape[1]
    full = lambda shape: pl.BlockSpec(shape, lambda i: (0,) * len(shape))
    return pl.pallas_call(
        _gcn_body,
        grid=(bs,),
        in_specs=[
            pl.BlockSpec((1, n, 2), lambda i: (i, 0, 0)),
            pl.BlockSpec((1, 2, n), lambda i: (i, 0, 0)),
            full(W1.shape),
            full((1, b1.shape[0])),
            full(W2.shape),
            full((1, b2.shape[0])),
            full(W3.shape),
            full((1, b3.shape[0])),
        ],
        out_specs=pl.BlockSpec((1, n, d3), lambda i: (i, 0, 0)),
        out_shape=jax.ShapeDtypeStruct((bs, n, d3), jnp.float32),
        compiler_params=pltpu.CompilerParams(
            dimension_semantics=("parallel",)),
    )(points, pt, W1, b1.reshape(1, -1), W2, b2.reshape(1, -1),
      W3, b3.reshape(1, -1))


# speedup vs baseline: 4624.6154x; 3.1598x over previous
import functools

import jax
import jax.numpy as jnp
from jax.experimental import pallas as pl
from jax.experimental.pallas import tpu as pltpu

_N = 1024
_PREC = jax.lax.Precision.DEFAULT


def _gcn_body(p_ref, pt_ref, w1_ref, b1_ref, w2_ref, b2_ref, w3_ref, b3_ref,
              out_ref):
    p = p_ref[0]
    pt = pt_ref[0]
    px_c = p[:, 0:1]
    py_c = p[:, 1:2]
    px_r = pt[0:1, :]
    py_r = pt[1:2, :]

    dx = px_c - px_r
    dy = py_c - py_r
    dist = jnp.sqrt(dx * dx + dy * dy)
    a = (dist < 1.0).astype(jnp.float32)

    deg_c = jnp.sum(a, axis=1, keepdims=True)
    deg_r = jnp.sum(a, axis=0, keepdims=True)
    r_c = 1.0 / jnp.sqrt(deg_c)
    r_r = 1.0 / jnp.sqrt(deg_r)
    nrm = a * r_c * r_r

    xw1 = px_c * w1_ref[0:1, :] + py_c * w1_ref[1:2, :]
    h1 = jax.nn.relu(
        jnp.dot(nrm, xw1, preferred_element_type=jnp.float32,
                precision=_PREC) + b1_ref[0:1, :])
    xw2 = jnp.dot(h1, w2_ref[...], preferred_element_type=jnp.float32,
                  precision=_PREC)
    h2 = jax.nn.relu(
        jnp.dot(nrm, xw2, preferred_element_type=jnp.float32,
                precision=_PREC) + b2_ref[0:1, :])
    xw3 = jnp.dot(h2, w3_ref[...], preferred_element_type=jnp.float32,
                  precision=_PREC)
    out_ref[0] = jnp.dot(nrm, xw3, preferred_element_type=jnp.float32,
                         precision=_PREC) + b3_ref[0:1, :]


@jax.jit
def kernel(points, W1, b1, W2, b2, W3, b3):
    bs, n, _ = points.shape
    pt = jnp.transpose(points, (0, 2, 1))
    d3 = W3.shape[1]
    full = lambda shape: pl.BlockSpec(shape, lambda i: (0,) * len(shape))
    return pl.pallas_call(
        _gcn_body,
        grid=(bs,),
        in_specs=[
            pl.BlockSpec((1, n, 2), lambda i: (i, 0, 0)),
            pl.BlockSpec((1, 2, n), lambda i: (i, 0, 0)),
            full(W1.shape),
            full((1, b1.shape[0])),
            full(W2.shape),
            full((1, b2.shape[0])),
            full(W3.shape),
            full((1, b3.shape[0])),
        ],
        out_specs=pl.BlockSpec((1, n, d3), lambda i: (i, 0, 0)),
        out_shape=jax.ShapeDtypeStruct((bs, n, d3), jnp.float32),
        compiler_params=pltpu.CompilerParams(
            dimension_semantics=("parallel",)),
    )(points, pt, W1, b1.reshape(1, -1), W2, b2.reshape(1, -1),
      W3, b3.reshape(1, -1))


# speedup vs baseline: 5187.2557x; 1.1217x over previous
import jax
import jax.numpy as jnp
from jax.experimental import pallas as pl
from jax.experimental.pallas import tpu as pltpu


def _gcn_body(p_ref, pt_ref, w1_ref, b1_ref, w2_ref, b2_ref, w3_ref, b3_ref,
              out_ref):
    p = p_ref[0]
    pt = pt_ref[0]
    px_c = p[:, 0:1]
    py_c = p[:, 1:2]
    px_r = pt[0:1, :]
    py_r = pt[1:2, :]

    dx = px_c - px_r
    dy = py_c - py_r
    a = (dx * dx + dy * dy < 1.0).astype(jnp.float32)

    deg_c = jnp.sum(a, axis=1, keepdims=True)
    deg_r = jnp.sum(a, axis=0, keepdims=True)
    r_c = jax.lax.rsqrt(deg_c)
    r_r = jax.lax.rsqrt(deg_r)
    nrm = (a * r_c * r_r).astype(jnp.bfloat16)

    f32 = jnp.float32
    xw1 = (px_c * w1_ref[0:1, :] + py_c * w1_ref[1:2, :]).astype(jnp.bfloat16)
    h1 = jax.nn.relu(
        jnp.dot(nrm, xw1, preferred_element_type=f32) + b1_ref[0:1, :])
    xw2 = jnp.dot(h1.astype(jnp.bfloat16), w2_ref[...],
                  preferred_element_type=f32).astype(jnp.bfloat16)
    h2 = jax.nn.relu(
        jnp.dot(nrm, xw2, preferred_element_type=f32) + b2_ref[0:1, :])
    xw3 = jnp.dot(h2.astype(jnp.bfloat16), w3_ref[...],
                  preferred_element_type=f32).astype(jnp.bfloat16)
    out_ref[0] = jnp.dot(nrm, xw3, preferred_element_type=f32) + b3_ref[0:1, :]


@jax.jit
def kernel(points, W1, b1, W2, b2, W3, b3):
    bs, n, _ = points.shape
    pt = jnp.transpose(points, (0, 2, 1))
    d3 = W3.shape[1]
    full = lambda shape: pl.BlockSpec(shape, lambda i: (0,) * len(shape))
    return pl.pallas_call(
        _gcn_body,
        grid=(bs,),
        in_specs=[
            pl.BlockSpec((1, n, 2), lambda i: (i, 0, 0)),
            pl.BlockSpec((1, 2, n), lambda i: (i, 0, 0)),
            full(W1.shape),
            full((1, b1.shape[0])),
            full(W2.shape),
            full((1, b2.shape[0])),
            full(W3.shape),
            full((1, b3.shape[0])),
        ],
        out_specs=pl.BlockSpec((1, n, d3), lambda i: (i, 0, 0)),
        out_shape=jax.ShapeDtypeStruct((bs, n, d3), jnp.float32),
        compiler_params=pltpu.CompilerParams(
            dimension_semantics=("parallel",)),
    )(points, pt, W1, b1.reshape(1, -1), W2, b2.reshape(1, -1),
      W3, b3.reshape(1, -1))
